# Initial kernel scaffold; baseline (speedup 1.0000x reference)
import jax
import jax.numpy as jnp
from jax.experimental import pallas as pl


def kernel(atom_type, pos, atomic_energy, emb_weight):
    raise NotImplementedError("write your pallas kernel here")


# speedup vs baseline: 19.5588x; 19.5588x over previous
import functools

import jax
import jax.numpy as jnp
from jax import lax
from jax.experimental import pallas as pl
from jax.experimental.pallas import tpu as pltpu
from jax.experimental.pallas import tpu_sc as plsc

_N = 100000
_NUM_TYPES = 100
_NC = 2
_NS = 16
_NW = _NC * _NS
_L = 16
_N_PAD = 100352
_B_PER_W = _N_PAD // _NW
_VECS = _B_PER_W // _L
_T_PAD = 128

_mesh = plsc.VectorSubcoreMesh(core_axis_name="c", subcore_axis_name="s")


@functools.partial(
    pl.kernel,
    out_type=jax.ShapeDtypeStruct((_N_PAD,), jnp.float32),
    mesh=_mesh,
    scratch_types=[
        pltpu.VMEM((_B_PER_W,), jnp.int32),
        pltpu.VMEM((_B_PER_W,), jnp.float32),
        pltpu.VMEM((_B_PER_W,), jnp.float32),
        pltpu.VMEM((_T_PAD,), jnp.float32),
    ],
    compiler_params=pltpu.CompilerParams(needs_layout_passes=False),
)
def _sc_embed_add(idx_hbm, en_hbm, tab_hbm, out_hbm, idx_v, en_v, out_v, tab_v):
    wid = lax.axis_index("s") * _NC + lax.axis_index("c")
    base = wid * _B_PER_W
    pltpu.sync_copy(tab_hbm, tab_v)
    pltpu.sync_copy(idx_hbm.at[pl.ds(base, _B_PER_W)], idx_v)
    pltpu.sync_copy(en_hbm.at[pl.ds(base, _B_PER_W)], en_v)

    def body(j, carry):
        off = j * _L
        iv = idx_v[pl.ds(off, _L)]
        g = plsc.load_gather(tab_v, [iv])
        out_v[pl.ds(off, _L)] = g + en_v[pl.ds(off, _L)]
        return carry

    lax.fori_loop(0, _VECS, body, 0)
    pltpu.sync_copy(out_v, out_hbm.at[pl.ds(base, _B_PER_W)])


def kernel(atom_type, pos, atomic_energy, emb_weight):
    idx = jnp.pad(atom_type.reshape(-1), (0, _N_PAD - _N))
    en = jnp.pad(atomic_energy.reshape(-1), (0, _N_PAD - _N))
    tab = jnp.pad(emb_weight.reshape(-1), (0, _T_PAD - _NUM_TYPES))
    out = _sc_embed_add(idx, en, tab)
    return out[:_N].reshape(_N, 1).astype(pos.dtype)


# speedup vs baseline: 20.4286x; 1.0445x over previous
import functools

import jax
import jax.numpy as jnp
from jax import lax
from jax.experimental import pallas as pl
from jax.experimental.pallas import tpu as pltpu
from jax.experimental.pallas import tpu_sc as plsc

_N = 100000
_NUM_TYPES = 100
_NC = 2
_NS = 16
_NW = _NC * _NS
_L = 16
_B_PER_W = 3136
_LAST_BASE = _N - _B_PER_W
_UNROLL = 4
_OUTER = _B_PER_W // (_L * _UNROLL)

_mesh = plsc.VectorSubcoreMesh(core_axis_name="c", subcore_axis_name="s")


@functools.partial(
    pl.kernel,
    out_type=jax.ShapeDtypeStruct((_N,), jnp.float32),
    mesh=_mesh,
    scratch_types=[
        pltpu.VMEM((_B_PER_W,), jnp.int32),
        pltpu.VMEM((_B_PER_W,), jnp.float32),
        pltpu.VMEM((_B_PER_W,), jnp.float32),
        pltpu.VMEM((_NUM_TYPES,), jnp.float32),
    ],
    compiler_params=pltpu.CompilerParams(needs_layout_passes=False),
)
def _sc_embed_add(idx_hbm, en_hbm, tab_hbm, out_hbm, idx_v, en_v, out_v, tab_v):
    wid = lax.axis_index("s") * _NC + lax.axis_index("c")
    base = lax.min(wid * _B_PER_W, _LAST_BASE)
    pltpu.sync_copy(tab_hbm, tab_v)
    pltpu.sync_copy(idx_hbm.at[pl.ds(base, _B_PER_W)], idx_v)
    pltpu.sync_copy(en_hbm.at[pl.ds(base, _B_PER_W)], en_v)

    def body(j, carry):
        off0 = j * (_L * _UNROLL)
        for b in range(_UNROLL):
            off = off0 + b * _L
            iv = idx_v[pl.ds(off, _L)]
            g = plsc.load_gather(tab_v, [iv])
            out_v[pl.ds(off, _L)] = g + en_v[pl.ds(off, _L)]
        return carry

    lax.fori_loop(0, _OUTER, body, 0)
    pltpu.sync_copy(out_v, out_hbm.at[pl.ds(base, _B_PER_W)])


def kernel(atom_type, pos, atomic_energy, emb_weight):
    out = _sc_embed_add(
        atom_type.reshape(-1),
        atomic_energy.reshape(-1),
        emb_weight.reshape(-1),
    )
    return out.reshape(_N, 1).astype(pos.dtype)


# speedup vs baseline: 21.9878x; 1.0763x over previous
import functools

import jax
import jax.numpy as jnp
from jax import lax
from jax.experimental import pallas as pl
from jax.experimental.pallas import tpu as pltpu
from jax.experimental.pallas import tpu_sc as plsc

_N = 100000
_NUM_TYPES = 100
_NC = 2
_NS = 16
_NW = _NC * _NS
_L = 16
_B_PER_W = 3136
_LAST_BASE = _N - _B_PER_W
_UNROLL = 4
_OUTER = _B_PER_W // _L

_mesh = plsc.VectorSubcoreMesh(core_axis_name="c", subcore_axis_name="s")


@functools.partial(
    pl.kernel,
    out_type=jax.ShapeDtypeStruct((_N,), jnp.float32),
    mesh=_mesh,
    scratch_types=[
        pltpu.VMEM((_B_PER_W,), jnp.int32),
        pltpu.VMEM((_B_PER_W,), jnp.float32),
        pltpu.VMEM((_B_PER_W,), jnp.float32),
        pltpu.VMEM((_NUM_TYPES,), jnp.float32),
        pltpu.SemaphoreType.DMA,
    ],
    compiler_params=pltpu.CompilerParams(needs_layout_passes=False),
)
def _sc_embed_add(idx_hbm, en_hbm, tab_hbm, out_hbm, idx_v, en_v, out_v, tab_v,
                  sem):
    wid = lax.axis_index("s") * _NC + lax.axis_index("c")
    base = lax.min(wid * _B_PER_W, _LAST_BASE)
    c1 = pltpu.async_copy(tab_hbm, tab_v, sem)
    c2 = pltpu.async_copy(idx_hbm.at[pl.ds(base, _B_PER_W)], idx_v, sem)
    c3 = pltpu.async_copy(en_hbm.at[pl.ds(base, _B_PER_W)], en_v, sem)
    c1.wait()
    c2.wait()
    c3.wait()

    @plsc.parallel_loop(0, _OUTER, 1, unroll=_UNROLL)
    def body(j):
        off = j * _L
        iv = idx_v[pl.ds(off, _L)]
        g = plsc.load_gather(tab_v, [iv])
        out_v[pl.ds(off, _L)] = g + en_v[pl.ds(off, _L)]

    pltpu.sync_copy(out_v, out_hbm.at[pl.ds(base, _B_PER_W)])


def kernel(atom_type, pos, atomic_energy, emb_weight):
    out = _sc_embed_add(
        atom_type.reshape(-1),
        atomic_energy.reshape(-1),
        emb_weight.reshape(-1),
    )
    return out.reshape(_N, 1).astype(pos.dtype)
